# baseline (speedup 1.0000x reference)
import functools

import jax
import jax.numpy as jnp
from jax import lax
from jax.experimental import pallas as pl
from jax.experimental.pallas import tpu as pltpu
from jax.experimental.pallas import tpu_sc as plsc

_CHUNK = 32
_WAVE = 4
_NBUF = 3


def _sc_gather_t(emb_t, pos_t, flat_idx, bn):
    d, v = emb_t.shape
    nt = flat_idx.shape[0]
    assert nt % _CHUNK == 0 and _CHUNK % bn == 0
    tpw = _CHUNK // bn
    n_active = nt // _CHUNK
    mesh = plsc.VectorSubcoreMesh(core_axis_name="c", subcore_axis_name="s")
    lanes = 16
    assert d % lanes == 0
    tail_w = v % 128
    tail_start = v - tail_w
    max_slab_g = v // 128 - 1
    n_waves = _CHUNK // _WAVE

    @functools.partial(
        pl.kernel,
        mesh=mesh,
        out_type=jax.ShapeDtypeStruct((nt, d), jnp.float32),
        scratch_types=[
            pltpu.VMEM((nt,), jnp.int32),
            pltpu.VMEM((_NBUF * _WAVE, d, 128), jnp.float32),
            pltpu.VMEM((d, max(tail_w, 1)), jnp.float32),
            pltpu.VMEM((d, 128), jnp.float32),
            pltpu.VMEM((_CHUNK, d), jnp.float32),
            pltpu.SemaphoreType.DMA,
            pltpu.SemaphoreType.DMA,
        ],
        compiler_params=pltpu.CompilerParams(needs_layout_passes=False),
    )
    def gather_kernel(
        table_hbm, pos_hbm, idx_hbm, out_hbm,
        idx_v, slabs_v, tail_v, pos_v, rows_v, sem, sem2,
    ):
        n_cores = lax.axis_size("c")
        wid = lax.axis_index("s") * n_cores + lax.axis_index("c")

        @pl.when(wid < n_active)
        def _():
            base = wid * _CHUNK
            pltpu.sync_copy(idx_hbm, idx_v)
            small_cps = [
                pltpu.async_copy(pos_hbm.at[:, pl.ds(0, 128)], pos_v, sem2)
            ]
            if tail_w:
                small_cps.append(
                    pltpu.async_copy(
                        table_hbm.at[:, pl.ds(tail_start, tail_w)],
                        tail_v, sem2,
                    )
                )
            lane_iota = lax.iota(jnp.int32, lanes)

            scalars, tvals = [], []
            for c in range(_CHUNK // lanes):
                chunk = idx_v[pl.ds(base + c * lanes, lanes)]
                for lane in range(lanes):
                    k = c * lanes + lane
                    scalars.append(
                        jnp.max(jnp.where(lane_iota == lane, chunk, 0))
                    )
                    tvals.append(wid * tpw + k // bn)
            for cp in small_cps:
                cp.wait()

            starts = [None] * _CHUNK
            wave_cps = [None] * n_waves

            def issue(wv):
                cps = []
                for i in range(_WAVE):
                    k = wv * _WAVE + i
                    s = scalars[k]
                    g = jnp.minimum(s // 128, max_slab_g)
                    start = pl.multiple_of(g * 128, 128)
                    starts[k] = start
                    cps.append(
                        pltpu.async_copy(
                            table_hbm.at[:, pl.ds(start, 128)],
                            slabs_v.at[(wv % _NBUF) * _WAVE + i],
                            sem,
                        )
                    )
                wave_cps[wv] = cps

            for wv in range(min(_NBUF, n_waves)):
                issue(wv)
            for wv in range(n_waves):
                for cp in wave_cps[wv]:
                    cp.wait()
                for i in range(_WAVE):
                    k = wv * _WAVE + i
                    s = scalars[k]
                    smod = s - starts[k]
                    col_m = jnp.full((lanes,), jnp.minimum(smod, 127))
                    kk_v = jnp.full(
                        (lanes,), (wv % _NBUF) * _WAVE + i, dtype=jnp.int32
                    )
                    t_col = jnp.full((lanes,), tvals[k])
                    if tail_w:
                        col_t = jnp.full(
                            (lanes,), jnp.clip(s - tail_start, 0, tail_w - 1)
                        )
                        sel_m = jnp.full((lanes,), smod < 128)
                    for c in range(d // lanes):
                        rowv = c * lanes + lane_iota
                        gm = plsc.load_gather(slabs_v, [kk_v, rowv, col_m])
                        if tail_w:
                            gt = plsc.load_gather(tail_v, [rowv, col_t])
                            res = jnp.where(sel_m, gm, gt)
                        else:
                            res = gm
                        res = res + plsc.load_gather(pos_v, [rowv, t_col])
                        rows_v[k, pl.ds(c * lanes, lanes)] = res
                if wv + _NBUF < n_waves:
                    issue(wv + _NBUF)
            pltpu.sync_copy(rows_v, out_hbm.at[pl.ds(base, _CHUNK)])

    return gather_kernel(emb_t, pos_t, flat_idx)


def _tc_body(x_ref, w_ref, b_ref, o_ref):
    y = lax.dot_general(
        x_ref[...], w_ref[...],
        dimension_numbers=(((1,), (0,)), ((), ())),
        preferred_element_type=jnp.float32,
    )
    o_ref[...] = y + b_ref[...][None, :]


def _tc_logits(x, w_t, b, tv):
    nt, d = x.shape
    v = w_t.shape[1]
    nv = (v + tv - 1) // tv
    return pl.pallas_call(
        _tc_body,
        grid=(nv,),
        in_specs=[
            pl.BlockSpec((nt, d), lambda j: (0, 0)),
            pl.BlockSpec((d, tv), lambda j: (0, j)),
            pl.BlockSpec((tv,), lambda j: (j,)),
        ],
        out_specs=pl.BlockSpec((nt, tv), lambda j: (0, j)),
        out_shape=jax.ShapeDtypeStruct((nt, v), jnp.float32),
    )(x, w_t, b)


def kernel(index, emb_table, pos_table, W, b):
    bn, tn = index.shape
    v, d = W.shape
    nt = bn * tn
    w_t = jnp.transpose(W)
    emb_t = jnp.transpose(emb_table)
    pos_t = jnp.transpose(pos_table)
    assert tn <= 128
    idx_tb = jnp.transpose(index).reshape(nt).astype(jnp.int32)
    x = _sc_gather_t(emb_t, pos_t, idx_tb, bn)
    logits = _tc_logits(x, w_t, b, tv=6144)
    return jnp.transpose(logits.reshape(tn, bn, v), (1, 0, 2))


# speedup vs baseline: 1.0012x; 1.0012x over previous
import functools

import jax
import jax.numpy as jnp
from jax import lax
from jax.experimental import pallas as pl
from jax.experimental.pallas import tpu as pltpu
from jax.experimental.pallas import tpu_sc as plsc

_CHUNK = 32
_WAVE = 4
_NBUF = 3


def _sc_gather_t(emb_t, pos_t, flat_idx, bn):
    d, v = emb_t.shape
    nt = flat_idx.shape[0]
    assert nt % _CHUNK == 0 and _CHUNK % bn == 0
    tpw = _CHUNK // bn
    n_active = nt // _CHUNK
    mesh = plsc.VectorSubcoreMesh(core_axis_name="c", subcore_axis_name="s")
    lanes = 16
    assert d % lanes == 0
    tail_w = v % 128
    tail_start = v - tail_w
    max_slab_g = v // 128 - 1
    n_waves = _CHUNK // _WAVE

    @functools.partial(
        pl.kernel,
        mesh=mesh,
        out_type=jax.ShapeDtypeStruct((nt, d), jnp.float32),
        scratch_types=[
            pltpu.VMEM((nt,), jnp.int32),
            pltpu.VMEM((_NBUF * _WAVE, d, 128), jnp.float32),
            pltpu.VMEM((d, max(tail_w, 1)), jnp.float32),
            pltpu.VMEM((d, 128), jnp.float32),
            pltpu.VMEM((_CHUNK, d), jnp.float32),
            pltpu.SemaphoreType.DMA,
            pltpu.SemaphoreType.DMA,
        ],
        compiler_params=pltpu.CompilerParams(needs_layout_passes=False),
    )
    def gather_kernel(
        table_hbm, pos_hbm, idx_hbm, out_hbm,
        idx_v, slabs_v, tail_v, pos_v, rows_v, sem, sem2,
    ):
        n_cores = lax.axis_size("c")
        wid = lax.axis_index("s") * n_cores + lax.axis_index("c")

        @pl.when(wid < n_active)
        def _():
            base = wid * _CHUNK
            pltpu.sync_copy(idx_hbm, idx_v)
            small_cps = [
                pltpu.async_copy(pos_hbm.at[:, pl.ds(0, 128)], pos_v, sem2)
            ]
            if tail_w:
                small_cps.append(
                    pltpu.async_copy(
                        table_hbm.at[:, pl.ds(tail_start, tail_w)],
                        tail_v, sem2,
                    )
                )
            lane_iota = lax.iota(jnp.int32, lanes)

            scalars, tvals = [], []
            for c in range(_CHUNK // lanes):
                chunk = idx_v[pl.ds(base + c * lanes, lanes)]
                for lane in range(lanes):
                    k = c * lanes + lane
                    scalars.append(
                        jnp.max(jnp.where(lane_iota == lane, chunk, 0))
                    )
                    tvals.append(wid * tpw + k // bn)
            for cp in small_cps:
                cp.wait()

            starts = [None] * _CHUNK
            wave_cps = [None] * n_waves

            def issue(wv):
                cps = []
                for i in range(_WAVE):
                    k = wv * _WAVE + i
                    s = scalars[k]
                    g = jnp.minimum(s // 128, max_slab_g)
                    start = pl.multiple_of(g * 128, 128)
                    starts[k] = start
                    cps.append(
                        pltpu.async_copy(
                            table_hbm.at[:, pl.ds(start, 128)],
                            slabs_v.at[(wv % _NBUF) * _WAVE + i],
                            sem,
                        )
                    )
                wave_cps[wv] = cps

            for wv in range(min(_NBUF, n_waves)):
                issue(wv)
            for wv in range(n_waves):
                for cp in wave_cps[wv]:
                    cp.wait()
                for i in range(_WAVE):
                    k = wv * _WAVE + i
                    s = scalars[k]
                    smod = s - starts[k]
                    col_m = jnp.full((lanes,), jnp.minimum(smod, 127))
                    kk_v = jnp.full(
                        (lanes,), (wv % _NBUF) * _WAVE + i, dtype=jnp.int32
                    )
                    t_col = jnp.full((lanes,), tvals[k])
                    if tail_w:
                        col_t = jnp.full(
                            (lanes,), jnp.clip(s - tail_start, 0, tail_w - 1)
                        )
                        sel_m = jnp.full((lanes,), smod < 128)
                    for c in range(d // lanes):
                        rowv = c * lanes + lane_iota
                        gm = plsc.load_gather(slabs_v, [kk_v, rowv, col_m])
                        if tail_w:
                            gt = plsc.load_gather(tail_v, [rowv, col_t])
                            res = jnp.where(sel_m, gm, gt)
                        else:
                            res = gm
                        res = res + plsc.load_gather(pos_v, [rowv, t_col])
                        rows_v[k, pl.ds(c * lanes, lanes)] = res
                if wv + _NBUF < n_waves:
                    issue(wv + _NBUF)
            pltpu.sync_copy(rows_v, out_hbm.at[pl.ds(base, _CHUNK)])

    return gather_kernel(emb_t, pos_t, flat_idx)


def _tc_body(x_ref, w_ref, b_ref, o_ref):
    y = lax.dot_general(
        x_ref[...], w_ref[...],
        dimension_numbers=(((1,), (0,)), ((), ())),
        preferred_element_type=jnp.float32,
    )
    o_ref[...] = y + b_ref[...][None, :]


def _tc_logits(x, w_t, b, tv):
    nt, d = x.shape
    v = w_t.shape[1]
    nv = (v + tv - 1) // tv
    return pl.pallas_call(
        _tc_body,
        grid=(nv,),
        in_specs=[
            pl.BlockSpec((nt, d), lambda j: (0, 0)),
            pl.BlockSpec((d, tv), lambda j: (0, j)),
            pl.BlockSpec((tv,), lambda j: (j,)),
        ],
        out_specs=pl.BlockSpec((nt, tv), lambda j: (0, j)),
        out_shape=jax.ShapeDtypeStruct((nt, v), jnp.float32),
    )(x, w_t, b)


def kernel(index, emb_table, pos_table, W, b):
    bn, tn = index.shape
    v, d = W.shape
    nt = bn * tn
    w_t = jnp.transpose(W)
    emb_t = jnp.transpose(emb_table)
    pos_t = jnp.transpose(pos_table)
    assert tn <= 128
    idx_tb = jnp.transpose(index).reshape(nt).astype(jnp.int32)
    x = _sc_gather_t(emb_t, pos_t, idx_tb, bn)
    logits = _tc_logits(x, w_t, b, tv=4096)
    return jnp.transpose(logits.reshape(tn, bn, v), (1, 0, 2))
